# Initial kernel scaffold; baseline (speedup 1.0000x reference)
import jax
import jax.numpy as jnp
from jax.experimental import pallas as pl


def kernel(s2, months, patch_size, input_res, channel_emb_s2):
    raise NotImplementedError("write your pallas kernel here")


# speedup vs baseline: 1.1798x; 1.1798x over previous
import functools
import math

import jax
import jax.numpy as jnp
from jax.experimental import pallas as pl
from jax.experimental.pallas import tpu as pltpu

_BASE_GSD = 10.0
_LN10K = math.log(10000.0)


def _body(s2_ref, months_ref, ch_ref, res_ref, out_ref, *, H, W, T, C, D):
    dq = D // 4
    f32 = jnp.float32
    i = pl.program_id(0)
    res = res_ref[0]

    def fiota(shape, dim):
        return jax.lax.broadcasted_iota(jnp.int32, shape, dim).astype(f32)

    k16 = fiota((T, dq // 2), 1)
    om16 = jnp.exp(k16 * (-_LN10K / (dq // 2)))
    tv = fiota((T, dq // 2), 0)
    ang_t = tv * om16
    pos_tab = jnp.concatenate([jnp.sin(ang_t), jnp.cos(ang_t)], axis=-1)

    mth = months_ref[0].astype(f32)
    ang_m = jnp.broadcast_to(mth * (2.0 * math.pi / 12.0), (T, dq // 2))
    m_emb = jnp.concatenate([jnp.sin(ang_m), jnp.cos(ang_m)], axis=-1)

    k8 = fiota((W, dq // 4), 1)
    om8 = jnp.exp(k8 * (-_LN10K / (dq // 4)))
    jv = fiota((W, dq // 4), 0) * res
    ang_w = jv * om8
    hf = (i % H).astype(f32) * res
    ang_h = hf * om8
    sp_blk = jnp.concatenate(
        [jnp.sin(ang_w), jnp.cos(ang_w), jnp.sin(ang_h), jnp.cos(ang_h)], axis=-1
    )

    ch = ch_ref[...]
    emb = jnp.concatenate(
        [
            jnp.broadcast_to(ch[None, None], (W, T, C, dq)),
            jnp.broadcast_to(pos_tab[None, :, None], (W, T, C, dq)),
            jnp.broadcast_to(m_emb[None, :, None], (W, T, C, dq)),
            jnp.broadcast_to(sp_blk[:, None, None], (W, T, C, dq)),
        ],
        axis=-1,
    )
    out_ref[0] = s2_ref[0] + emb


def kernel(s2, months, patch_size, input_res, channel_emb_s2):
    b, h, w, t, c_g, d = s2.shape
    res = (jnp.asarray(input_res, jnp.float32) * patch_size / _BASE_GSD).reshape(1)
    s2v = s2.reshape(b * h, w, t, c_g, d)
    months3 = months.reshape(b, t, 1)

    body = functools.partial(_body, H=h, W=w, T=t, C=c_g, D=d)
    out = pl.pallas_call(
        body,
        grid=(b * h,),
        in_specs=[
            pl.BlockSpec((1, w, t, c_g, d), lambda i: (i, 0, 0, 0, 0)),
            pl.BlockSpec((1, t, 1), lambda i, _h=h: (i // _h, 0, 0)),
            pl.BlockSpec((c_g, d // 4), lambda i: (0, 0)),
            pl.BlockSpec(memory_space=pltpu.SMEM),
        ],
        out_specs=pl.BlockSpec((1, w, t, c_g, d), lambda i: (i, 0, 0, 0, 0)),
        out_shape=jax.ShapeDtypeStruct(s2v.shape, s2.dtype),
    )(s2v, months3, channel_emb_s2, res)
    return out.reshape(s2.shape)


# speedup vs baseline: 1.4719x; 1.2476x over previous
import functools
import math

import jax
import jax.numpy as jnp
from jax.experimental import pallas as pl
from jax.experimental.pallas import tpu as pltpu

_BASE_GSD = 10.0
_LN10K = math.log(10000.0)


def _body(s2_ref, months_ref, ch_ref, res_ref, out_ref, *, H, W, T, C, D):
    dq = D // 4
    f32 = jnp.float32
    i = pl.program_id(0)
    res = res_ref[0]

    def fiota(shape, dim):
        return jax.lax.broadcasted_iota(jnp.int32, shape, dim).astype(f32)

    k16 = fiota((T, dq // 2), 1)
    om16 = jnp.exp(k16 * (-_LN10K / (dq // 2)))
    tv = fiota((T, dq // 2), 0)
    ang_t = tv * om16
    pos_tab = jnp.concatenate([jnp.sin(ang_t), jnp.cos(ang_t)], axis=-1)

    mth = months_ref[0].astype(f32)
    ang_m = jnp.broadcast_to(mth * (2.0 * math.pi / 12.0), (T, dq // 2))
    m_emb = jnp.concatenate([jnp.sin(ang_m), jnp.cos(ang_m)], axis=-1)

    k8 = fiota((W, dq // 4), 1)
    om8 = jnp.exp(k8 * (-_LN10K / (dq // 4)))
    jv = fiota((W, dq // 4), 0) * res
    ang_w = jv * om8
    hf = (i % H).astype(f32) * res
    ang_h = hf * om8
    sp_blk = jnp.concatenate(
        [jnp.sin(ang_w), jnp.cos(ang_w), jnp.sin(ang_h), jnp.cos(ang_h)], axis=-1
    )

    ch = ch_ref[...]
    base = jnp.concatenate(
        [
            jnp.broadcast_to(ch[None], (T, C, dq)).reshape(T * C, dq),
            jnp.broadcast_to(pos_tab[:, None], (T, C, dq)).reshape(T * C, dq),
            jnp.broadcast_to(m_emb[:, None], (T, C, dq)).reshape(T * C, dq),
            jnp.zeros((T * C, dq), f32),
        ],
        axis=-1,
    )
    spfull = jnp.concatenate([jnp.zeros((W, 3 * dq), f32), sp_blk], axis=-1)
    out_ref[0] = s2_ref[0] + base[None] + spfull[:, None, :]


def kernel(s2, months, patch_size, input_res, channel_emb_s2):
    b, h, w, t, c_g, d = s2.shape
    res = (jnp.asarray(input_res, jnp.float32) * patch_size / _BASE_GSD).reshape(1)
    s2v = s2.reshape(b * h, w, t * c_g, d)
    months3 = months.reshape(b, t, 1)

    body = functools.partial(_body, H=h, W=w, T=t, C=c_g, D=d)
    out = pl.pallas_call(
        body,
        grid=(b * h,),
        in_specs=[
            pl.BlockSpec((1, w, t * c_g, d), lambda i: (i, 0, 0, 0)),
            pl.BlockSpec((1, t, 1), lambda i, _h=h: (i // _h, 0, 0)),
            pl.BlockSpec((c_g, d // 4), lambda i: (0, 0)),
            pl.BlockSpec(memory_space=pltpu.SMEM),
        ],
        out_specs=pl.BlockSpec((1, w, t * c_g, d), lambda i: (i, 0, 0, 0)),
        out_shape=jax.ShapeDtypeStruct(s2v.shape, s2.dtype),
    )(s2v, months3, channel_emb_s2, res)
    return out.reshape(s2.shape)


# speedup vs baseline: 1.6833x; 1.1437x over previous
import functools
import math

import jax
import jax.numpy as jnp
from jax.experimental import pallas as pl
from jax.experimental.pallas import tpu as pltpu

_BASE_GSD = 10.0
_LN10K = math.log(10000.0)


def _body(s2_ref, months_ref, ch_ref, res_ref, out_ref, *, H, W, T, C, D, RH):
    dq = D // 4
    f32 = jnp.float32
    i = pl.program_id(0)
    res = res_ref[0]

    def fiota(shape, dim):
        return jax.lax.broadcasted_iota(jnp.int32, shape, dim).astype(f32)

    k16 = fiota((T, dq // 2), 1)
    om16 = jnp.exp(k16 * (-_LN10K / (dq // 2)))
    tv = fiota((T, dq // 2), 0)
    ang_t = tv * om16
    pos_tab = jnp.concatenate([jnp.sin(ang_t), jnp.cos(ang_t)], axis=-1)

    mth = months_ref[0].astype(f32)
    ang_m = jnp.broadcast_to(mth * (2.0 * math.pi / 12.0), (T, dq // 2))
    m_emb = jnp.concatenate([jnp.sin(ang_m), jnp.cos(ang_m)], axis=-1)

    ch = ch_ref[...]
    base = jnp.concatenate(
        [
            jnp.broadcast_to(ch[None], (T, C, dq)).reshape(T * C, dq),
            jnp.broadcast_to(pos_tab[:, None], (T, C, dq)).reshape(T * C, dq),
            jnp.broadcast_to(m_emb[:, None], (T, C, dq)).reshape(T * C, dq),
            jnp.zeros((T * C, dq), f32),
        ],
        axis=-1,
    )

    k8 = fiota((W, dq // 4), 1)
    om8 = jnp.exp(k8 * (-_LN10K / (dq // 4)))
    jv = fiota((W, dq // 4), 0) * res
    ang_w = jv * om8
    emb_w = jnp.concatenate([jnp.sin(ang_w), jnp.cos(ang_w)], axis=-1)
    hbase = (i * RH) % H
    k8r = fiota((RH, dq // 4), 1)
    om8r = jnp.exp(k8r * (-_LN10K / (dq // 4)))
    hv = (fiota((RH, dq // 4), 0) + hbase.astype(f32)) * res
    ang_h = hv * om8r
    emb_h = jnp.concatenate([jnp.sin(ang_h), jnp.cos(ang_h)], axis=-1)
    sp = jnp.concatenate(
        [
            jnp.broadcast_to(emb_w[None], (RH, W, dq // 2)),
            jnp.broadcast_to(emb_h[:, None], (RH, W, dq // 2)),
        ],
        axis=-1,
    )
    spfull = jnp.concatenate([jnp.zeros((RH, W, 3 * dq), f32), sp], axis=-1)

    out_ref[...] = s2_ref[...] + base[None, None] + spfull[:, :, None, :]


def kernel(s2, months, patch_size, input_res, channel_emb_s2):
    b, h, w, t, c_g, d = s2.shape
    RH = 2
    res = (jnp.asarray(input_res, jnp.float32) * patch_size / _BASE_GSD).reshape(1)
    s2v = s2.reshape(b * h, w, t * c_g, d)
    months3 = months.reshape(b, t, 1)
    steps_per_b = h // RH

    body = functools.partial(_body, H=h, W=w, T=t, C=c_g, D=d, RH=RH)
    out = pl.pallas_call(
        body,
        grid=(b * h // RH,),
        in_specs=[
            pl.BlockSpec((RH, w, t * c_g, d), lambda i: (i, 0, 0, 0)),
            pl.BlockSpec((1, t, 1), lambda i, _s=steps_per_b: (i // _s, 0, 0)),
            pl.BlockSpec((c_g, d // 4), lambda i: (0, 0)),
            pl.BlockSpec(memory_space=pltpu.SMEM),
        ],
        out_specs=pl.BlockSpec((RH, w, t * c_g, d), lambda i: (i, 0, 0, 0)),
        out_shape=jax.ShapeDtypeStruct(s2v.shape, s2.dtype),
    )(s2v, months3, channel_emb_s2, res)
    return out.reshape(s2.shape)


# speedup vs baseline: 1.7554x; 1.0428x over previous
import functools
import math

import jax
import jax.numpy as jnp
from jax.experimental import pallas as pl
from jax.experimental.pallas import tpu as pltpu

_BASE_GSD = 10.0
_LN10K = math.log(10000.0)


def _body(s2_ref, months_ref, ch_ref, res_ref, out_ref, *, H, W, T, C, D, RH):
    dq = D // 4
    f32 = jnp.float32
    i = pl.program_id(0)
    res = res_ref[0]

    def fiota(shape, dim):
        return jax.lax.broadcasted_iota(jnp.int32, shape, dim).astype(f32)

    k16 = fiota((T, dq // 2), 1)
    om16 = jnp.exp(k16 * (-_LN10K / (dq // 2)))
    tv = fiota((T, dq // 2), 0)
    ang_t = tv * om16
    pos_tab = jnp.concatenate([jnp.sin(ang_t), jnp.cos(ang_t)], axis=-1)

    mth = months_ref[0].astype(f32)
    ang_m = jnp.broadcast_to(mth * (2.0 * math.pi / 12.0), (T, dq // 2))
    m_emb = jnp.concatenate([jnp.sin(ang_m), jnp.cos(ang_m)], axis=-1)

    ch = ch_ref[...]
    base = jnp.concatenate(
        [
            jnp.broadcast_to(ch[None], (T, C, dq)).reshape(T * C, dq),
            jnp.broadcast_to(pos_tab[:, None], (T, C, dq)).reshape(T * C, dq),
            jnp.broadcast_to(m_emb[:, None], (T, C, dq)).reshape(T * C, dq),
            jnp.zeros((T * C, dq), f32),
        ],
        axis=-1,
    )

    k8 = fiota((W, dq // 4), 1)
    om8 = jnp.exp(k8 * (-_LN10K / (dq // 4)))
    jv = fiota((W, dq // 4), 0) * res
    ang_w = jv * om8
    emb_w = jnp.concatenate([jnp.sin(ang_w), jnp.cos(ang_w)], axis=-1)
    hbase = (i * RH) % H
    k8r = fiota((RH, dq // 4), 1)
    om8r = jnp.exp(k8r * (-_LN10K / (dq // 4)))
    hv = (fiota((RH, dq // 4), 0) + hbase.astype(f32)) * res
    ang_h = hv * om8r
    emb_h = jnp.concatenate([jnp.sin(ang_h), jnp.cos(ang_h)], axis=-1)
    sp = jnp.concatenate(
        [
            jnp.broadcast_to(emb_w[None], (RH, W, dq // 2)),
            jnp.broadcast_to(emb_h[:, None], (RH, W, dq // 2)),
        ],
        axis=-1,
    )
    spfull = jnp.concatenate([jnp.zeros((RH, W, 3 * dq), f32), sp], axis=-1)

    out_ref[...] = s2_ref[...] + base[None, None] + spfull[:, :, None, :]


def kernel(s2, months, patch_size, input_res, channel_emb_s2):
    b, h, w, t, c_g, d = s2.shape
    RH = 4
    res = (jnp.asarray(input_res, jnp.float32) * patch_size / _BASE_GSD).reshape(1)
    s2v = s2.reshape(b * h, w, t * c_g, d)
    months3 = months.reshape(b, t, 1)
    steps_per_b = h // RH

    body = functools.partial(_body, H=h, W=w, T=t, C=c_g, D=d, RH=RH)
    out = pl.pallas_call(
        body,
        grid=(b * h // RH,),
        in_specs=[
            pl.BlockSpec((RH, w, t * c_g, d), lambda i: (i, 0, 0, 0)),
            pl.BlockSpec((1, t, 1), lambda i, _s=steps_per_b: (i // _s, 0, 0)),
            pl.BlockSpec((c_g, d // 4), lambda i: (0, 0)),
            pl.BlockSpec(memory_space=pltpu.SMEM),
        ],
        out_specs=pl.BlockSpec((RH, w, t * c_g, d), lambda i: (i, 0, 0, 0)),
        out_shape=jax.ShapeDtypeStruct(s2v.shape, s2.dtype),
    )(s2v, months3, channel_emb_s2, res)
    return out.reshape(s2.shape)
